# Initial kernel scaffold; baseline (speedup 1.0000x reference)
import jax
import jax.numpy as jnp
from jax.experimental import pallas as pl


def kernel(s, col_id, out_nbrs, E_bias, rms_w, cW1, cb1, cW2, cb2, qW1, qb1, qW2, qb2, kW1, kb1, kW2, kb2):
    raise NotImplementedError("write your pallas kernel here")


# speedup vs baseline: 4.3888x; 4.3888x over previous
import functools

import jax
import jax.numpy as jnp
from jax import lax
from jax.experimental import pallas as pl
from jax.experimental.pallas import tpu as pltpu
from jax.experimental.pallas import tpu_sc as plsc

_N = 50000
_K = 16
_DS = 128
_DID = 128
_FF = 256
_HQ = 64
_EPS = 1e-6


_BLK = 400
_GRID = _N // _BLK


def _gelu(x):
    return 0.5 * x * (1.0 + lax.erf(x * 0.7071067811865476))


def _dense_body(s_ref, cid_ref, rmsw_ref, cW1_ref, cb1_ref, cW2_ref, cb2_ref,
                qW1_ref, qb1_ref, qW2_ref, qb2_ref, kW1_ref, kb1_ref,
                kW2_ref, kb2_ref, m_ref, q_ref, k_ref):
    s = s_ref[...]
    cid = cid_ref[...]
    s_n = s * lax.rsqrt(jnp.mean(s * s, axis=-1, keepdims=True) + _EPS)
    s_n = s_n * rmsw_ref[...]
    x = jnp.concatenate([s_n, cid], axis=-1)
    h = _gelu(jnp.dot(x, cW1_ref[...], preferred_element_type=jnp.float32)
              + cb1_ref[...])
    m_ref[...] = (jnp.dot(h, cW2_ref[...], preferred_element_type=jnp.float32)
                  + cb2_ref[...])
    hq = _gelu(jnp.dot(x, qW1_ref[...], preferred_element_type=jnp.float32)
               + qb1_ref[...])
    q_ref[...] = (jnp.dot(hq, qW2_ref[...], preferred_element_type=jnp.float32)
                  + qb2_ref[...])
    hk = _gelu(jnp.dot(cid, kW1_ref[...], preferred_element_type=jnp.float32)
               + kb1_ref[...])
    k_ref[...] = (jnp.dot(hk, kW2_ref[...], preferred_element_type=jnp.float32)
                  + kb2_ref[...])


def _full(shape):
    nd = len(shape)
    return pl.BlockSpec(shape, lambda i, _nd=nd: (0,) * _nd)


def _dense(s, cid, rms_w, cW1, cb1, cW2, cb2, qW1, qb1, qW2, qb2,
           kW1, kb1, kW2, kb2):
    row_spec = lambda w: pl.BlockSpec((_BLK, w), lambda i: (i, 0))
    return pl.pallas_call(
        _dense_body,
        grid=(_GRID,),
        in_specs=[
            row_spec(_DS), row_spec(_DID),
            _full((1, _DS)),
            _full((_DS + _DID, _FF)), _full((1, _FF)),
            _full((_FF, _DS)), _full((1, _DS)),
            _full((_DS + _DID, 2 * _HQ)), _full((1, 2 * _HQ)),
            _full((2 * _HQ, _HQ)), _full((1, _HQ)),
            _full((_DID, 2 * _HQ)), _full((1, 2 * _HQ)),
            _full((2 * _HQ, _HQ)), _full((1, _HQ)),
        ],
        out_specs=[row_spec(_DS), row_spec(_HQ), row_spec(_HQ)],
        out_shape=[
            jax.ShapeDtypeStruct((_N, _DS), jnp.float32),
            jax.ShapeDtypeStruct((_N, _HQ), jnp.float32),
            jax.ShapeDtypeStruct((_N, _HQ), jnp.float32),
        ],
    )(s, cid, rms_w.reshape(1, _DS), cW1, cb1.reshape(1, _FF),
      cW2, cb2.reshape(1, _DS), qW1, qb1.reshape(1, 2 * _HQ),
      qW2, qb2.reshape(1, _HQ), kW1, kb1.reshape(1, 2 * _HQ),
      kW2, kb2.reshape(1, _HQ))


_NW = 32
_NPAD = 50176
_ROWS_W = _NPAD // _NW
_R = 32
_CHUNKS = _ROWS_W // _R
_IDX_ROWS = _R * _K // 128
_IDX_W = _ROWS_W * _K // 128

_sc_mesh = plsc.VectorSubcoreMesh(core_axis_name="c", subcore_axis_name="s")


@functools.partial(
    pl.kernel,
    mesh=_sc_mesh,
    compiler_params=pltpu.CompilerParams(
        needs_layout_passes=False, use_tc_tiling_on_sc=False),
    out_type=jax.ShapeDtypeStruct((_NPAD, _K), jnp.float32),
    scratch_types=[
        pltpu.VMEM((_IDX_ROWS, 128), jnp.int32),
        pltpu.VMEM((_R * _K, _HQ), jnp.float32),
        pltpu.VMEM((_R, _HQ), jnp.float32),
        pltpu.VMEM((_R, _K), jnp.float32),
        pltpu.VMEM((_R, _K), jnp.float32),
        pltpu.SemaphoreType.DMA,
    ],
)
def _sc_score(q_hbm, k_hbm, idx_hbm, eb_hbm, out_hbm,
              idx_v, kn_v, q_v, eb_v, o_v, sem):
    wid = lax.axis_index("s") * 2 + lax.axis_index("c")
    lane = lax.iota(jnp.int32, 16)

    def chunk_body(ci, carry):
        base = wid * _ROWS_W + ci * _R
        ib = wid * _IDX_W + ci * _IDX_ROWS
        pltpu.sync_copy(idx_hbm.at[pl.ds(ib, _IDX_ROWS)], idx_v)
        copies = []
        for j in range(_IDX_ROWS):
            copies.append(pltpu.async_copy(
                k_hbm.at[idx_v.at[j]], kn_v.at[pl.ds(j * 128, 128)], sem))
        pltpu.sync_copy(q_hbm.at[pl.ds(base, _R)], q_v)
        pltpu.sync_copy(eb_hbm.at[pl.ds(base, _R)], eb_v)
        for cp in copies:
            cp.wait()

        def row_body(r, c2):
            rows = r * _K + lane
            acc = jnp.zeros((16,), jnp.float32)
            for dg in range(_HQ // 16):
                qv = q_v[r, pl.ds(dg * 16, 16)]
                for j in range(16):
                    d = dg * 16 + j
                    col = jnp.full((16,), d, jnp.int32)
                    g = plsc.load_gather(kn_v, [rows, col])
                    acc = acc + g * qv[j]
            score = acc + eb_v[r]
            o_v[r] = 1.0 / (1.0 + jnp.exp(-score))
            return c2

        lax.fori_loop(0, _R, row_body, 0)
        pltpu.sync_copy(o_v, out_hbm.at[pl.ds(base, _R)])
        return carry

    lax.fori_loop(0, _CHUNKS, chunk_body, 0)


def kernel(s, col_id, out_nbrs, E_bias, rms_w, cW1, cb1, cW2, cb2,
           qW1, qb1, qW2, qb2, kW1, kb1, kW2, kb2):
    m_out, q, k = _dense(s, col_id, rms_w, cW1, cb1, cW2, cb2,
                         qW1, qb1, qW2, qb2, kW1, kb1, kW2, kb2)
    pad = _NPAD - _N
    idx = jnp.pad(out_nbrs.astype(jnp.int32), ((0, pad), (0, 0)))
    idx2d = idx.reshape(_NPAD * _K // 128, 128)
    q_pad = jnp.pad(q, ((0, pad), (0, 0)))
    eb_pad = jnp.pad(E_bias, ((0, pad), (0, 0)))
    w_pad = _sc_score(q_pad, k, idx2d, eb_pad)
    return m_out, w_pad[:_N]
